# baseline (speedup 1.0000x reference)
import functools

import jax
import jax.numpy as jnp
from jax import lax
from jax.experimental import pallas as pl
from jax.experimental.pallas import tpu as pltpu
from jax.experimental.pallas import tpu_sc as plsc

NUM_EMB = 1000000
D = 64
B = 4096
H = 50

NC = 2
NS = 16
NW = NC * NS
L = 16

BPW = B // NW
BPC = 2
CROWS = BPC * H
NCHUNK = BPW // BPC
NBUF = 2
ND = D // L

_mesh = plsc.VectorSubcoreMesh(core_axis_name="c", subcore_axis_name="s")


@functools.partial(
    pl.kernel,
    out_type=jax.ShapeDtypeStruct((B, D), jnp.float32),
    mesh=_mesh,
    scratch_types=[
        pltpu.VMEM((NCHUNK, CROWS), jnp.int32),
        pltpu.VMEM((NBUF, CROWS, D), jnp.float32),
        pltpu.VMEM((BPW, D), jnp.float32),
        pltpu.SemaphoreType.DMA,
        pltpu.SemaphoreType.DMA,
    ],
    compiler_params=pltpu.CompilerParams(use_tc_tiling_on_sc=False),
)
def _embbag(x2d, table, out, idx_v, ring_v, out_v, sem0, sem1):
    sems = (sem0, sem1)
    w = lax.axis_index("c") * NS + lax.axis_index("s")
    inv = jnp.float32(1.0 / H)

    pltpu.sync_copy(x2d.at[pl.ds(w * NCHUNK, NCHUNK), :], idx_v)

    for b in range(NBUF):
        pltpu.make_async_copy(table.at[idx_v.at[b]], ring_v.at[b], sems[b]).start()

    def step(i, carry):
        for b in range(NBUF):
            g = i * NBUF + b
            pltpu.make_async_copy(
                table.at[idx_v.at[g]], ring_v.at[b], sems[b]).wait()
            for bb in range(BPC):
                r0 = bb * H
                accs = [[ring_v[b, r0 + k, pl.ds(L * d, L)] for k in range(2)]
                        for d in range(ND)]
                for j in range(2, H, 2):
                    for d in range(ND):
                        for k in range(2):
                            accs[d][k] = accs[d][k] + ring_v[
                                b, r0 + j + k, pl.ds(L * d, L)]
                row = g * BPC + bb
                for d in range(ND):
                    out_v[row, pl.ds(L * d, L)] = (accs[d][0] + accs[d][1]) * inv
            ng = g + NBUF

            @pl.when(ng < NCHUNK)
            def _start(ng=ng, b=b):
                pltpu.make_async_copy(
                    table.at[idx_v.at[ng]], ring_v.at[b], sems[b]).start()
        return carry

    lax.fori_loop(0, NCHUNK // NBUF, step, 0)

    pltpu.sync_copy(out_v, out.at[pl.ds(w * BPW, BPW), :])


def kernel(x, table):
    x2d = x.reshape(B * H // CROWS, CROWS).astype(jnp.int32)
    return _embbag(x2d, table)


# speedup vs baseline: 1.5714x; 1.5714x over previous
import functools

import jax
import jax.numpy as jnp
from jax import lax
from jax.experimental import pallas as pl
from jax.experimental.pallas import tpu as pltpu
from jax.experimental.pallas import tpu_sc as plsc

NUM_EMB = 1000000
D = 64
DP = 128
B = 4096
H = 50

NC = 2
NS = 16
NW = NC * NS
L = 16

BPW = B // NW
BPC = 2
CROWS = BPC * H
NCHUNK = BPW // BPC
NBUF = 2
ND = D // L

_mesh = plsc.VectorSubcoreMesh(core_axis_name="c", subcore_axis_name="s")


@functools.partial(
    pl.kernel,
    out_type=jax.ShapeDtypeStruct((B, D), jnp.float32),
    mesh=_mesh,
    scratch_types=[
        pltpu.VMEM((NCHUNK, CROWS), jnp.int32),
        pltpu.VMEM((NBUF, CROWS, DP), jnp.float32),
        pltpu.VMEM((BPW, D), jnp.float32),
        pltpu.SemaphoreType.DMA,
        pltpu.SemaphoreType.DMA,
    ],
    compiler_params=pltpu.CompilerParams(use_tc_tiling_on_sc=False),
)
def _embbag(x2d, table, out, idx_v, ring_v, out_v, sem0, sem1):
    sems = (sem0, sem1)
    w = lax.axis_index("c") * NS + lax.axis_index("s")
    inv = jnp.float32(1.0 / H)

    pltpu.sync_copy(x2d.at[pl.ds(w * NCHUNK, NCHUNK), :], idx_v)

    for b in range(NBUF):
        pltpu.make_async_copy(table.at[idx_v.at[b]], ring_v.at[b], sems[b]).start()

    def step(i, carry):
        for b in range(NBUF):
            g = i * NBUF + b
            pltpu.make_async_copy(
                table.at[idx_v.at[g]], ring_v.at[b], sems[b]).wait()
            for bb in range(BPC):
                r0 = bb * H
                accs = [[ring_v[b, r0 + k, pl.ds(L * d, L)] for k in range(2)]
                        for d in range(ND)]
                for j in range(2, H, 2):
                    for d in range(ND):
                        for k in range(2):
                            accs[d][k] = accs[d][k] + ring_v[
                                b, r0 + j + k, pl.ds(L * d, L)]
                row = g * BPC + bb
                for d in range(ND):
                    out_v[row, pl.ds(L * d, L)] = (accs[d][0] + accs[d][1]) * inv
            ng = g + NBUF

            @pl.when(ng < NCHUNK)
            def _start(ng=ng, b=b):
                pltpu.make_async_copy(
                    table.at[idx_v.at[ng]], ring_v.at[b], sems[b]).start()
        return carry

    lax.fori_loop(0, NCHUNK // NBUF, step, 0)

    pltpu.sync_copy(out_v, out.at[pl.ds(w * BPW, BPW), :])


_BLK = 4096


def _relayout_body(tin_ref, tout_ref):
    t = tin_ref[...]
    tout_ref[...] = jnp.concatenate(
        [t.T, jnp.zeros((_BLK, DP - D), jnp.float32)], axis=1)


_relayout = pl.pallas_call(
    _relayout_body,
    grid=(pl.cdiv(NUM_EMB, _BLK),),
    in_specs=[pl.BlockSpec((D, _BLK), lambda i: (0, i))],
    out_specs=pl.BlockSpec((_BLK, DP), lambda i: (i, 0)),
    out_shape=jax.ShapeDtypeStruct((NUM_EMB, DP), jnp.float32),
)


def kernel(x, table):
    x2d = x.reshape(B * H // CROWS, CROWS).astype(jnp.int32)
    tp = _relayout(table.T)
    return _embbag(x2d, tp)
